# Initial kernel scaffold; baseline (speedup 1.0000x reference)
import jax
import jax.numpy as jnp
from jax.experimental import pallas as pl


def kernel(pred, target):
    raise NotImplementedError("write your pallas kernel here")


# speedup vs baseline: 1.7997x; 1.7997x over previous
import functools

import jax
import jax.numpy as jnp
from jax.experimental import pallas as pl


def _chamfer_body(pred_ref, target_ref, out_ref, *, n, m, block_n):
    t = target_ref[...]
    tn = jnp.sum(t * t, axis=1, keepdims=True).T

    def body(i, carry):
        col_min, row_sum = carry
        p = pred_ref[pl.ds(i * block_n, block_n), :]
        pn = jnp.sum(p * p, axis=1, keepdims=True)
        cross = jax.lax.dot_general(
            p, t, (((1,), (1,)), ((), ())),
            preferred_element_type=jnp.float32)
        d2 = pn + tn - 2.0 * cross
        row_min = jnp.min(d2, axis=1, keepdims=True)
        row_sum = row_sum + jnp.sum(
            jnp.sqrt(jnp.maximum(row_min, 0.0) + 1e-12))
        col_min = jnp.minimum(col_min, jnp.min(d2, axis=0, keepdims=True))
        return col_min, row_sum

    col_min, row_sum = jax.lax.fori_loop(
        0, n // block_n, body,
        (jnp.full((1, m), jnp.inf, dtype=jnp.float32),
         jnp.zeros((1, 1), dtype=jnp.float32)))
    back = jnp.sum(jnp.sqrt(jnp.maximum(col_min, 0.0) + 1e-12),
                   axis=1, keepdims=True)
    out_ref[...] = (row_sum / n + back / m) * 0.5


def kernel(pred, target):
    pred = pred.astype(jnp.float32)
    target = target.astype(jnp.float32)
    n, k = pred.shape
    m, _ = target.shape
    pred_p = jnp.pad(pred, ((0, 0), (0, 8 - k)))
    target_p = jnp.pad(target, ((0, 0), (0, 8 - k)))
    out = pl.pallas_call(
        functools.partial(_chamfer_body, n=n, m=m, block_n=256),
        out_shape=jax.ShapeDtypeStruct((1, 1), jnp.float32),
    )(pred_p, target_p)
    return out[0, 0]


# speedup vs baseline: 2.3650x; 1.3141x over previous
import functools

import jax
import jax.numpy as jnp
from jax.experimental import pallas as pl


def _chamfer_body(pred_ref, predt_ref, targett_ref, out_ref, *, n, m, block_n):
    tt = targett_ref[...]
    tn = tt[0:1, :] * tt[0:1, :] + tt[1:2, :] * tt[1:2, :] \
        + tt[2:3, :] * tt[2:3, :]

    def body(i, carry):
        col_min, row_sum = carry
        p = pred_ref[pl.ds(i * block_n, block_n), :]
        pn = jnp.sum(p * p, axis=1, keepdims=True)
        pblk = -2.0 * predt_ref[:, pl.ds(i * block_n, block_n)]
        cross = jax.lax.dot_general(
            pblk, tt, (((0,), (0,)), ((), ())),
            preferred_element_type=jnp.float32)
        d2 = (pn + tn) + cross
        row_min = jnp.min(d2, axis=1, keepdims=True)
        row_sum = row_sum + jnp.sum(
            jnp.sqrt(jnp.maximum(row_min, 0.0) + 1e-12))
        col_min = jnp.minimum(col_min, jnp.min(d2, axis=0, keepdims=True))
        return col_min, row_sum

    col_min, row_sum = jax.lax.fori_loop(
        0, n // block_n, body,
        (jnp.full((1, m), jnp.inf, dtype=jnp.float32),
         jnp.zeros((1, 1), dtype=jnp.float32)))
    back = jnp.sum(jnp.sqrt(jnp.maximum(col_min, 0.0) + 1e-12),
                   axis=1, keepdims=True)
    out_ref[...] = (row_sum / n + back / m) * 0.5


def kernel(pred, target):
    pred = pred.astype(jnp.float32)
    target = target.astype(jnp.float32)
    n, k = pred.shape
    m, _ = target.shape
    pred_rows = jnp.pad(pred, ((0, 0), (0, 8 - k)))
    predt = jnp.pad(pred.T, ((0, 8 - k), (0, 0)))
    targett = jnp.pad(target.T, ((0, 8 - k), (0, 0)))
    out = pl.pallas_call(
        functools.partial(_chamfer_body, n=n, m=m, block_n=256),
        out_shape=jax.ShapeDtypeStruct((1, 1), jnp.float32),
    )(pred_rows, predt, targett)
    return out[0, 0]
